# baseline (speedup 1.0000x reference)
import functools

import jax
import jax.numpy as jnp
from jax import lax
from jax.experimental import pallas as pl
from jax.experimental.pallas import tpu as pltpu
from jax.experimental.pallas import tpu_sc as plsc

N = 19
E = 342
LATENT = 512
HID = 2048
T = 4096

SC_NC = 2
SC_NS = 16
E_PAD = 512
EPW = E_PAD // (SC_NC * SC_NS)
N_PAD = 24
SCRAP = 20


def _sc_agg(z, src_pad, dst_pad):
    mesh = plsc.VectorSubcoreMesh(core_axis_name="c", subcore_axis_name="s")
    nw = SC_NC * SC_NS
    tile_w = N_PAD * LATENT

    @functools.partial(
        pl.kernel, mesh=mesh,
        compiler_params=pltpu.CompilerParams(needs_layout_passes=False),
        out_type=jax.ShapeDtypeStruct((nw * tile_w,), jnp.float32),
        scratch_types=[
            pltpu.VMEM((EPW,), jnp.int32),
            pltpu.VMEM((EPW,), jnp.int32),
            pltpu.VMEM((EPW, LATENT), jnp.float32),
            pltpu.VMEM((tile_w,), jnp.float32),
            pltpu.SemaphoreType.DMA,
        ],
    )
    def k(z_hbm, src_hbm, dst_hbm, out_hbm, src_v, dst_v, rows_v, acc_v, sem):
        cid = lax.axis_index("c")
        sid = lax.axis_index("s")
        wid = sid * SC_NC + cid
        base = wid * EPW
        pltpu.sync_copy(src_hbm.at[pl.ds(base, EPW)], src_v)
        pltpu.sync_copy(dst_hbm.at[pl.ds(base, EPW)], dst_v)
        pltpu.async_copy(z_hbm.at[src_v], rows_v, sem).wait()
        zero16 = jnp.zeros((16,), jnp.float32)

        @pl.loop(0, tile_w // 16)
        def _zero(i):
            acc_v[pl.ds(i * 16, 16)] = zero16

        dst16 = dst_v[...]
        lane = jax.lax.broadcasted_iota(jnp.int32, (16,), 0)

        @pl.loop(0, LATENT // 16)
        def _cols(c):
            for r in range(16):
                cols = c * 16 + ((lane + r) & 15)
                x = plsc.load_gather(rows_v, [lane, cols])
                plsc.addupdate_scatter(acc_v, [dst16 * LATENT + cols], x)
        pltpu.sync_copy(acc_v, out_hbm.at[pl.ds(wid * tile_w, tile_w)])

    return k(z, src_pad, dst_pad)


HI = jax.lax.Precision.HIGHEST

P2 = 1
P3 = P2 + 4
P4 = P3 + 8
P5 = P4 + 8
NSTEP = P5 + N


def _adj(e_ref):
    e = e_ref[...]
    src = e[0:1, :]
    dst = e[1:2, :]
    ii = jax.lax.broadcasted_iota(jnp.int32, (N, E), 0)
    d1h = (ii == dst).astype(jnp.float32)
    s1h = (ii == src).astype(jnp.float32)
    return jax.lax.dot_general(d1h, s1h, (((1,), (1,)), ((), ())),
                               preferred_element_type=jnp.float32,
                               precision=HI)


def _body(agg_ref, e_ref, z_ref, w1a_ref, b1a_ref, w1b_ref, b1b_ref,
          w2a_ref, b2a_ref, w2b_ref, b2b_ref, wc1_ref, bc1_ref, wc2_ref,
          bc2_ref, recon_ref, pred_ref,
          h_ref, x2_ref, g_ref, acc1_ref, acc2_ref, acc3_ref, accl_ref):
    s = pl.program_id(0)

    @pl.when(s == 0)
    def _p1():
        ag = agg_ref[...].reshape(SC_NC * SC_NS, N_PAD, LATENT)
        x1 = z_ref[...] + jnp.sum(ag, axis=0)[:N, :]
        h = jnp.maximum(jnp.dot(x1, w1a_ref[...]) + b1a_ref[...], 0.0)
        for i in range(4):
            h_ref[i] = h[:, i * 512:(i + 1) * 512]

    @pl.when((s >= P2) & (s < P3))
    def _p2():
        k = s - P2
        part = jnp.dot(h_ref[pl.ds(k, 1)][0], w1b_ref[...])

        @pl.when(k == 0)
        def _():
            acc1_ref[...] = part

        @pl.when(k > 0)
        def _():
            acc1_ref[...] += part

    @pl.when(s == P3)
    def _p3a():
        h1 = jnp.maximum(acc1_ref[...] + b1b_ref[...], 0.0)
        a = _adj(e_ref)
        x2 = h1 + jnp.dot(a, h1, precision=HI)
        for i in range(8):
            x2_ref[i] = x2[:, i * 256:(i + 1) * 256]

    @pl.when((s >= P3) & (s < P4))
    def _p3():
        k = s - P3
        part = jnp.dot(x2_ref[pl.ds(k, 1)][0], w2a_ref[...])

        @pl.when(k == 0)
        def _():
            acc2_ref[...] = part

        @pl.when(k > 0)
        def _():
            acc2_ref[...] += part

    @pl.when(s == P4)
    def _p4a():
        g = jnp.maximum(acc2_ref[...] + b2a_ref[...], 0.0)
        for i in range(8):
            g_ref[i] = g[:, i * 512:(i + 1) * 512]

    @pl.when((s >= P4) & (s < P5))
    def _p4():
        k = s - P4
        part = jnp.dot(g_ref[pl.ds(k, 1)][0], w2b_ref[...])

        @pl.when(k == 0)
        def _():
            acc3_ref[...] = part

        @pl.when(k > 0)
        def _():
            acc3_ref[...] += part

        @pl.when(k == 7)
        def _():
            acc3_ref[...] += b2b_ref[...]
            recon_ref[...] = acc3_ref[...]

    @pl.when(s >= P5)
    def _p5():
        n = s - P5
        sel = (jax.lax.broadcasted_iota(jnp.int32, (1, N), 1)
               == n).astype(jnp.float32)
        row = jnp.dot(sel, acc3_ref[...], precision=HI)
        part = jnp.dot(row, wc1_ref[...])

        @pl.when(n == 0)
        def _():
            accl_ref[...] = part

        @pl.when(n > 0)
        def _():
            accl_ref[...] += part

        @pl.when(n == N - 1)
        def _():
            logits = accl_ref[...] + bc1_ref[...]
            pred_ref[...] = jax.nn.sigmoid(
                jnp.dot(logits, wc2_ref[...]) + bc2_ref[...])


def kernel(latent_z, edge_idx, W1a, b1a, W1b, b1b, W2a, b2a, W2b, b2b, Wc1, bc1, Wc2, bc2):
    e = edge_idx.astype(jnp.int32)
    npad = E_PAD - E
    src_pad = jnp.concatenate([e[0], jnp.zeros((npad,), jnp.int32)])
    dst_pad = jnp.concatenate([e[1], jnp.full((npad,), SCRAP, jnp.int32)])
    agg = _sc_agg(latent_z, src_pad, dst_pad).reshape(SC_NC * SC_NS * N_PAD, LATENT)

    def pin(shape):
        return pl.BlockSpec(shape, lambda s: (0, 0))

    in_specs = [
        pl.BlockSpec((SC_NC * SC_NS * N_PAD, LATENT), lambda s: (0, 0)),
        pin((2, E)),
        pin((N, LATENT)),
        pin((LATENT, HID)),
        pin((1, HID)),
        pl.BlockSpec((512, HID), lambda s: (jnp.clip(s - P2, 0, 3), 0)),
        pin((1, HID)),
        pl.BlockSpec((256, T), lambda s: (jnp.clip(s - P3, 0, 7), 0)),
        pin((1, T)),
        pl.BlockSpec((512, T), lambda s: (jnp.clip(s - P4, 0, 7), 0)),
        pin((1, T)),
        pl.BlockSpec((T, LATENT), lambda s: (jnp.clip(s - P5, 0, N - 1), 0)),
        pin((1, LATENT)),
        pin((LATENT, 1)),
        pin((1, 1)),
    ]
    out_specs = [pin((N, T)), pin((1, 1))]
    out_shape = [jax.ShapeDtypeStruct((N, T), jnp.float32),
                 jax.ShapeDtypeStruct((1, 1), jnp.float32)]
    scratch = [
        pltpu.VMEM((4, N, 512), jnp.float32),
        pltpu.VMEM((8, N, 256), jnp.float32),
        pltpu.VMEM((8, N, 512), jnp.float32),
        pltpu.VMEM((N, HID), jnp.float32),
        pltpu.VMEM((N, T), jnp.float32),
        pltpu.VMEM((N, T), jnp.float32),
        pltpu.VMEM((1, LATENT), jnp.float32),
    ]

    recon, pred = pl.pallas_call(
        _body,
        grid=(NSTEP,),
        in_specs=in_specs,
        out_specs=out_specs,
        out_shape=out_shape,
        scratch_shapes=scratch,
        compiler_params=pltpu.CompilerParams(
            dimension_semantics=("arbitrary",)),
    )(agg, e, latent_z, W1a, b1a.reshape(1, HID), W1b, b1b.reshape(1, HID),
      W2a, b2a.reshape(1, T), W2b, b2b.reshape(1, T),
      Wc1, bc1.reshape(1, LATENT), Wc2, bc2.reshape(1, 1))

    return (pred.reshape(1), recon)


# speedup vs baseline: 1.0118x; 1.0118x over previous
import functools

import jax
import jax.numpy as jnp
from jax import lax
from jax.experimental import pallas as pl
from jax.experimental.pallas import tpu as pltpu
from jax.experimental.pallas import tpu_sc as plsc

N = 19
E = 342
LATENT = 512
HID = 2048
T = 4096

SC_NC = 2
SC_NS = 16
E_PAD = 512
EPW = E_PAD // (SC_NC * SC_NS)
N_PAD = 24
SCRAP = 20


def _sc_agg(z, src_pad, dst_pad):
    mesh = plsc.VectorSubcoreMesh(core_axis_name="c", subcore_axis_name="s")
    nw = SC_NC * SC_NS
    tile_w = N_PAD * LATENT

    @functools.partial(
        pl.kernel, mesh=mesh,
        compiler_params=pltpu.CompilerParams(needs_layout_passes=False),
        out_type=jax.ShapeDtypeStruct((nw * tile_w,), jnp.float32),
        scratch_types=[
            pltpu.VMEM((EPW,), jnp.int32),
            pltpu.VMEM((EPW,), jnp.int32),
            pltpu.VMEM((EPW, LATENT), jnp.float32),
            pltpu.VMEM((tile_w,), jnp.float32),
            pltpu.SemaphoreType.DMA,
        ],
    )
    def k(z_hbm, src_hbm, dst_hbm, out_hbm, src_v, dst_v, rows_v, acc_v, sem):
        cid = lax.axis_index("c")
        sid = lax.axis_index("s")
        wid = sid * SC_NC + cid
        base = wid * EPW
        pltpu.sync_copy(src_hbm.at[pl.ds(base, EPW)], src_v)
        pltpu.sync_copy(dst_hbm.at[pl.ds(base, EPW)], dst_v)
        pltpu.async_copy(z_hbm.at[src_v], rows_v, sem).wait()
        zero16 = jnp.zeros((16,), jnp.float32)

        @pl.loop(0, tile_w // 16, unroll=16)
        def _zero(i):
            acc_v[pl.ds(i * 16, 16)] = zero16

        dst16 = dst_v[...]
        lane = jax.lax.broadcasted_iota(jnp.int32, (16,), 0)

        @pl.loop(0, LATENT // 16, unroll=4)
        def _cols(c):
            for r in range(16):
                cols = c * 16 + ((lane + r) & 15)
                x = plsc.load_gather(rows_v, [lane, cols])
                plsc.addupdate_scatter(acc_v, [dst16 * LATENT + cols], x)
        pltpu.sync_copy(acc_v, out_hbm.at[pl.ds(wid * tile_w, tile_w)])

    return k(z, src_pad, dst_pad)


HI = jax.lax.Precision.HIGHEST

P2 = 1
P3 = P2 + 4
P4 = P3 + 8
P5 = P4 + 8
NSTEP = P5 + N


def _adj(e_ref):
    e = e_ref[...]
    src = e[0:1, :]
    dst = e[1:2, :]
    ii = jax.lax.broadcasted_iota(jnp.int32, (N, E), 0)
    d1h = (ii == dst).astype(jnp.float32)
    s1h = (ii == src).astype(jnp.float32)
    return jax.lax.dot_general(d1h, s1h, (((1,), (1,)), ((), ())),
                               preferred_element_type=jnp.float32,
                               precision=HI)


def _body(agg_ref, e_ref, z_ref, w1a_ref, b1a_ref, w1b_ref, b1b_ref,
          w2a_ref, b2a_ref, w2b_ref, b2b_ref, wc1_ref, bc1_ref, wc2_ref,
          bc2_ref, recon_ref, pred_ref,
          h_ref, x2_ref, g_ref, acc1_ref, acc2_ref, acc3_ref, accl_ref):
    s = pl.program_id(0)

    @pl.when(s == 0)
    def _p1():
        ag = agg_ref[...].reshape(SC_NC * SC_NS, N_PAD, LATENT)
        x1 = z_ref[...] + jnp.sum(ag, axis=0)[:N, :]
        h = jnp.maximum(jnp.dot(x1, w1a_ref[...]) + b1a_ref[...], 0.0)
        for i in range(4):
            h_ref[i] = h[:, i * 512:(i + 1) * 512]

    @pl.when((s >= P2) & (s < P3))
    def _p2():
        k = s - P2
        part = jnp.dot(h_ref[pl.ds(k, 1)][0], w1b_ref[...])

        @pl.when(k == 0)
        def _():
            acc1_ref[...] = part

        @pl.when(k > 0)
        def _():
            acc1_ref[...] += part

    @pl.when(s == P3)
    def _p3a():
        h1 = jnp.maximum(acc1_ref[...] + b1b_ref[...], 0.0)
        a = _adj(e_ref)
        x2 = h1 + jnp.dot(a, h1, precision=HI)
        for i in range(8):
            x2_ref[i] = x2[:, i * 256:(i + 1) * 256]

    @pl.when((s >= P3) & (s < P4))
    def _p3():
        k = s - P3
        part = jnp.dot(x2_ref[pl.ds(k, 1)][0], w2a_ref[...])

        @pl.when(k == 0)
        def _():
            acc2_ref[...] = part

        @pl.when(k > 0)
        def _():
            acc2_ref[...] += part

    @pl.when(s == P4)
    def _p4a():
        g = jnp.maximum(acc2_ref[...] + b2a_ref[...], 0.0)
        for i in range(8):
            g_ref[i] = g[:, i * 512:(i + 1) * 512]

    @pl.when((s >= P4) & (s < P5))
    def _p4():
        k = s - P4
        part = jnp.dot(g_ref[pl.ds(k, 1)][0], w2b_ref[...])

        @pl.when(k == 0)
        def _():
            acc3_ref[...] = part

        @pl.when(k > 0)
        def _():
            acc3_ref[...] += part

        @pl.when(k == 7)
        def _():
            acc3_ref[...] += b2b_ref[...]
            recon_ref[...] = acc3_ref[...]

    @pl.when(s >= P5)
    def _p5():
        n = s - P5
        sel = (jax.lax.broadcasted_iota(jnp.int32, (1, N), 1)
               == n).astype(jnp.float32)
        row = jnp.dot(sel, acc3_ref[...], precision=HI)
        part = jnp.dot(row, wc1_ref[...])

        @pl.when(n == 0)
        def _():
            accl_ref[...] = part

        @pl.when(n > 0)
        def _():
            accl_ref[...] += part

        @pl.when(n == N - 1)
        def _():
            logits = accl_ref[...] + bc1_ref[...]
            pred_ref[...] = jax.nn.sigmoid(
                jnp.dot(logits, wc2_ref[...]) + bc2_ref[...])


def kernel(latent_z, edge_idx, W1a, b1a, W1b, b1b, W2a, b2a, W2b, b2b, Wc1, bc1, Wc2, bc2):
    e = edge_idx.astype(jnp.int32)
    npad = E_PAD - E
    src_pad = jnp.concatenate([e[0], jnp.zeros((npad,), jnp.int32)])
    dst_pad = jnp.concatenate([e[1], jnp.full((npad,), SCRAP, jnp.int32)])
    agg = _sc_agg(latent_z, src_pad, dst_pad).reshape(SC_NC * SC_NS * N_PAD, LATENT)

    def pin(shape):
        return pl.BlockSpec(shape, lambda s: (0, 0))

    in_specs = [
        pl.BlockSpec((SC_NC * SC_NS * N_PAD, LATENT), lambda s: (0, 0)),
        pin((2, E)),
        pin((N, LATENT)),
        pin((LATENT, HID)),
        pin((1, HID)),
        pl.BlockSpec((512, HID), lambda s: (jnp.clip(s - P2, 0, 3), 0)),
        pin((1, HID)),
        pl.BlockSpec((256, T), lambda s: (jnp.clip(s - P3, 0, 7), 0)),
        pin((1, T)),
        pl.BlockSpec((512, T), lambda s: (jnp.clip(s - P4, 0, 7), 0)),
        pin((1, T)),
        pl.BlockSpec((T, LATENT), lambda s: (jnp.clip(s - P5, 0, N - 1), 0)),
        pin((1, LATENT)),
        pin((LATENT, 1)),
        pin((1, 1)),
    ]
    out_specs = [pin((N, T)), pin((1, 1))]
    out_shape = [jax.ShapeDtypeStruct((N, T), jnp.float32),
                 jax.ShapeDtypeStruct((1, 1), jnp.float32)]
    scratch = [
        pltpu.VMEM((4, N, 512), jnp.float32),
        pltpu.VMEM((8, N, 256), jnp.float32),
        pltpu.VMEM((8, N, 512), jnp.float32),
        pltpu.VMEM((N, HID), jnp.float32),
        pltpu.VMEM((N, T), jnp.float32),
        pltpu.VMEM((N, T), jnp.float32),
        pltpu.VMEM((1, LATENT), jnp.float32),
    ]

    recon, pred = pl.pallas_call(
        _body,
        grid=(NSTEP,),
        in_specs=in_specs,
        out_specs=out_specs,
        out_shape=out_shape,
        scratch_shapes=scratch,
        compiler_params=pltpu.CompilerParams(
            dimension_semantics=("arbitrary",)),
    )(agg, e, latent_z, W1a, b1a.reshape(1, HID), W1b, b1b.reshape(1, HID),
      W2a, b2a.reshape(1, T), W2b, b2b.reshape(1, T),
      Wc1, bc1.reshape(1, LATENT), Wc2, bc2.reshape(1, 1))

    return (pred.reshape(1), recon)


# speedup vs baseline: 1.3741x; 1.3580x over previous
import jax
import jax.numpy as jnp
from jax.experimental import pallas as pl
from jax.experimental.pallas import tpu as pltpu

N = 19
E = 342
LATENT = 512
HID = 2048
T = 4096

HI = jax.lax.Precision.HIGHEST

P2 = 1
P3 = P2 + 4
P4 = P3 + 8
P5 = P4 + 8
NSTEP = P5 + N


def _adj(e_ref):
    e = e_ref[...]
    src = e[0:1, :]
    dst = e[1:2, :]
    ii = jax.lax.broadcasted_iota(jnp.int32, (N, E), 0)
    d1h = (ii == dst).astype(jnp.float32)
    s1h = (ii == src).astype(jnp.float32)
    return jax.lax.dot_general(d1h, s1h, (((1,), (1,)), ((), ())),
                               preferred_element_type=jnp.float32,
                               precision=HI)


def _body(e_ref, z_ref, w1a_ref, b1a_ref, w1b_ref, b1b_ref, w2a_ref, b2a_ref,
          w2b_ref, b2b_ref, wc1_ref, bc1_ref, wc2_ref, bc2_ref,
          recon_ref, pred_ref,
          h_ref, x2_ref, g_ref, rows_ref, acc1_ref, acc2_ref, acc3_ref,
          accl_ref):
    s = pl.program_id(0)

    @pl.when(s == 0)
    def _p1():
        a = _adj(e_ref)
        z = z_ref[...]
        x1 = z + jnp.dot(a, z, precision=HI)
        h = jnp.maximum(jnp.dot(x1, w1a_ref[...]) + b1a_ref[...], 0.0)
        for i in range(4):
            h_ref[i] = h[:, i * 512:(i + 1) * 512]

    @pl.when((s >= P2) & (s < P3))
    def _p2():
        k = s - P2
        part = jnp.dot(h_ref[pl.ds(k, 1)][0], w1b_ref[...])

        @pl.when(k == 0)
        def _():
            acc1_ref[...] = part

        @pl.when(k > 0)
        def _():
            acc1_ref[...] += part

    @pl.when(s == P3)
    def _p3a():
        h1 = jnp.maximum(acc1_ref[...] + b1b_ref[...], 0.0)
        a = _adj(e_ref)
        x2 = h1 + jnp.dot(a, h1, precision=HI)
        for i in range(8):
            x2_ref[i] = x2[:, i * 256:(i + 1) * 256]

    @pl.when((s >= P3) & (s < P4))
    def _p3():
        k = s - P3
        part = jnp.dot(x2_ref[pl.ds(k, 1)][0], w2a_ref[...])

        @pl.when(k == 0)
        def _():
            acc2_ref[...] = part

        @pl.when(k > 0)
        def _():
            acc2_ref[...] += part

    @pl.when(s == P4)
    def _p4a():
        g = jnp.maximum(acc2_ref[...] + b2a_ref[...], 0.0)
        for i in range(8):
            g_ref[i] = g[:, i * 512:(i + 1) * 512]

    @pl.when((s >= P4) & (s < P5))
    def _p4():
        k = s - P4
        part = jnp.dot(g_ref[pl.ds(k, 1)][0], w2b_ref[...])

        @pl.when(k == 0)
        def _():
            acc3_ref[...] = part

        @pl.when(k > 0)
        def _():
            acc3_ref[...] += part

        @pl.when(k == 7)
        def _():
            acc3_ref[...] += b2b_ref[...]
            r = acc3_ref[...]
            recon_ref[...] = r
            for n in range(N):
                rows_ref[n] = r[n:n + 1, :]

    @pl.when(s >= P5)
    def _p5():
        n = s - P5
        row = rows_ref[pl.ds(n, 1)][0]
        wcb = wc1_ref[...].astype(jnp.bfloat16)
        part = jax.lax.dot_general(
            row, wcb, (((1,), (0,)), ((), ())),
            preferred_element_type=jnp.float32)

        @pl.when(n == 0)
        def _():
            accl_ref[...] = part

        @pl.when(n > 0)
        def _():
            accl_ref[...] += part

        @pl.when(n == N - 1)
        def _():
            logits = accl_ref[...] + bc1_ref[...]
            pred_ref[...] = jax.nn.sigmoid(
                jnp.dot(logits, wc2_ref[...], precision=HI)
                + bc2_ref[...])


def kernel(latent_z, edge_idx, W1a, b1a, W1b, b1b, W2a, b2a, W2b, b2b, Wc1, bc1, Wc2, bc2):
    e = edge_idx.astype(jnp.int32)

    def pin(shape):
        return pl.BlockSpec(shape, lambda s: (0, 0))

    in_specs = [
        pin((2, E)),
        pin((N, LATENT)),
        pin((LATENT, HID)),
        pin((1, HID)),
        pl.BlockSpec((512, HID), lambda s: (jnp.clip(s - P2, 0, 3), 0)),
        pin((1, HID)),
        pl.BlockSpec((256, T), lambda s: (jnp.clip(s - P3, 0, 7), 0)),
        pin((1, T)),
        pl.BlockSpec((512, T), lambda s: (jnp.clip(s - P4, 0, 7), 0)),
        pin((1, T)),
        pl.BlockSpec((T, LATENT), lambda s: (jnp.clip(s - P5, 0, N - 1), 0)),
        pin((1, LATENT)),
        pin((LATENT, 1)),
        pin((1, 1)),
    ]
    out_specs = [pin((N, T)), pin((1, 1))]
    out_shape = [jax.ShapeDtypeStruct((N, T), jnp.float32),
                 jax.ShapeDtypeStruct((1, 1), jnp.float32)]
    scratch = [
        pltpu.VMEM((4, N, 512), jnp.float32),
        pltpu.VMEM((8, N, 256), jnp.float32),
        pltpu.VMEM((8, N, 512), jnp.float32),
        pltpu.VMEM((N, 1, T), jnp.float32),
        pltpu.VMEM((N, HID), jnp.float32),
        pltpu.VMEM((N, T), jnp.float32),
        pltpu.VMEM((N, T), jnp.float32),
        pltpu.VMEM((1, LATENT), jnp.float32),
    ]

    recon, pred = pl.pallas_call(
        _body,
        grid=(NSTEP,),
        in_specs=in_specs,
        out_specs=out_specs,
        out_shape=out_shape,
        scratch_shapes=scratch,
        compiler_params=pltpu.CompilerParams(
            dimension_semantics=("arbitrary",)),
    )(e, latent_z, W1a, b1a.reshape(1, HID), W1b, b1b.reshape(1, HID),
      W2a, b2a.reshape(1, T), W2b, b2b.reshape(1, T),
      Wc1, bc1.reshape(1, LATENT), Wc2, bc2.reshape(1, 1))

    return (pred.reshape(1), recon)
